# baseline (speedup 1.0000x reference)
import functools

import jax
import jax.numpy as jnp
from jax import lax
from jax.experimental import pallas as pl
from jax.experimental.pallas import tpu as pltpu
from jax.experimental.pallas import tpu_sc as plsc

_V = 1000
_D = 1000
_DP = 1024
_NSMP = 4096
_T = 20
_NC, _NS = 2, 16
_NW = _NC * _NS
_NDB = 8
_NBB = 4
_BB = _NSMP // _NBB
_BH = _BB // 2
_RC = 32
_DPS = 1025


def _sigmoid_body(x_ref, o_ref):
    x = x_ref[...]
    o_ref[...] = 1.0 / (1.0 + jnp.exp(-x))


def _sigmoid_table(logits_padded):
    return pl.pallas_call(
        _sigmoid_body,
        out_shape=jax.ShapeDtypeStruct(logits_padded.shape, jnp.float32),
    )(logits_padded)


_mesh = plsc.VectorSubcoreMesh(core_axis_name="c", subcore_axis_name="s")


@functools.partial(
    pl.kernel,
    out_type=jax.ShapeDtypeStruct((_T, _D, _NSMP), jnp.float32),
    mesh=_mesh,
    scratch_types=[
        pltpu.VMEM((_RC * _DPS,), jnp.float32),
        pltpu.VMEM((_T, _BB), jnp.int32),
        pltpu.VMEM((_RC, _BH), jnp.float32),
        pltpu.VMEM((_RC, _BH), jnp.float32),
        pltpu.SemaphoreType.DMA,
        pltpu.SemaphoreType.DMA,
    ],
    compiler_params=pltpu.CompilerParams(needs_layout_passes=False),
)
def _tgather_kernel(tt_hbm, idxt_hbm, out_hbm, ttbuf, idxbuf, ob0, ob1,
                    os0, os1):
    wid = lax.axis_index("s") * _NC + lax.axis_index("c")
    dblk = wid // _NBB
    dbase = dblk * 128
    b0 = (wid % _NBB) * _BB

    pltpu.sync_copy(idxt_hbm.at[:, pl.ds(b0, _BB)], idxbuf)

    obufs = (ob0, ob1)
    osems = (os0, os1)

    def out_wait(h, rows):
        pltpu.make_async_copy(
            obufs[h].at[pl.ds(0, rows)],
            out_hbm.at[0, pl.ds(0, rows), pl.ds(0, _BH)],
            osems[h],
        ).wait()

    def do_chunk(d0, rows):
        pltpu.sync_copy(
            tt_hbm.at[pl.ds(d0 * _DPS, rows * _DPS)],
            ttbuf.at[pl.ds(0, rows * _DPS)],
        )

        @pl.loop(0, 2 * _T, step=2)
        def _(k0):
            for h in range(2):
                k = k0 + h
                t = k // 2
                bh = b0 + (k % 2) * _BH
                ob = obufs[h]

                @pl.when(k0 >= 2)
                def _():
                    out_wait(h, rows)

                @pl.loop(0, _BH // 16, unroll=2)
                def _(g):
                    iv = idxbuf[t, pl.ds((k % 2) * _BH + g * 16, 16)]
                    for d0 in range(0, rows, 16):
                        w = min(16, rows - d0)
                        vals = [
                            plsc.load_gather(ttbuf, [iv + (d0 + j) * _DPS])
                            for j in range(w)
                        ]
                        for j in range(w):
                            ob[d0 + j, pl.ds(g * 16, 16)] = vals[j]

                pltpu.make_async_copy(
                    ob.at[pl.ds(0, rows)],
                    out_hbm.at[t, pl.ds(d0, rows), pl.ds(bh, _BH)],
                    osems[h],
                ).start()

        out_wait(0, rows)
        out_wait(1, rows)

    @pl.loop(0, 3)
    def _(c):
        do_chunk(dbase + c * _RC, _RC)

    @pl.when(dblk < _NDB - 1)
    def _():
        do_chunk(dbase + 3 * _RC, _RC)

    @pl.when(dblk == _NDB - 1)
    def _():
        do_chunk(dbase + 3 * _RC, 8)


def kernel(idx, morphisms_logits):
    table = _sigmoid_table(morphisms_logits)
    tt_flat = jnp.pad(jnp.transpose(table), ((0, 0), (0, _DPS - _V))).reshape(-1)
    idxt = jnp.transpose(idx.astype(jnp.int32))
    out_t = _tgather_kernel(tt_flat, idxt)
    return jnp.transpose(out_t, (2, 0, 1))


# speedup vs baseline: 1.0011x; 1.0011x over previous
import functools

import jax
import jax.numpy as jnp
from jax import lax
from jax.experimental import pallas as pl
from jax.experimental.pallas import tpu as pltpu
from jax.experimental.pallas import tpu_sc as plsc

_V = 1000
_D = 1000
_NSMP = 4096
_T = 20
_NC, _NS = 2, 16
_NW = _NC * _NS
_NDB = 8
_NBB = 4
_BB = _NSMP // _NBB
_BH = _BB // 2
_RC = 32
_DPS = 1025


def _sigmoid_body(x_ref, o_ref):
    x = x_ref[...]
    o_ref[...] = 1.0 / (1.0 + jnp.exp(-x))


def _sigmoid_table(logits_padded):
    return pl.pallas_call(
        _sigmoid_body,
        out_shape=jax.ShapeDtypeStruct(logits_padded.shape, jnp.float32),
    )(logits_padded)


_mesh = plsc.VectorSubcoreMesh(core_axis_name="c", subcore_axis_name="s")


@functools.partial(
    pl.kernel,
    out_type=jax.ShapeDtypeStruct((_T, _D, _NSMP), jnp.float32),
    mesh=_mesh,
    scratch_types=[
        pltpu.VMEM((_RC * _DPS,), jnp.float32),
        pltpu.VMEM((_T, _BB), jnp.int32),
        pltpu.VMEM((_RC, _BH), jnp.float32),
        pltpu.VMEM((_RC, _BH), jnp.float32),
        pltpu.SemaphoreType.DMA,
        pltpu.SemaphoreType.DMA,
    ],
    compiler_params=pltpu.CompilerParams(needs_layout_passes=False),
)
def _tgather_kernel(tt_hbm, idxt_hbm, out_hbm, ttbuf, idxbuf, ob0, ob1,
                    os0, os1):
    wid = lax.axis_index("s") * _NC + lax.axis_index("c")
    dblk = wid // _NBB
    dbase = dblk * 128
    b0 = (wid % _NBB) * _BB

    pltpu.sync_copy(idxt_hbm.at[:, pl.ds(b0, _BB)], idxbuf)

    obufs = (ob0, ob1)
    osems = (os0, os1)

    def out_wait(h, rows):
        pltpu.make_async_copy(
            obufs[h].at[pl.ds(0, rows)],
            out_hbm.at[0, pl.ds(0, rows), pl.ds(0, _BH)],
            osems[h],
        ).wait()

    def do_chunk(d0, rows):
        pltpu.sync_copy(
            tt_hbm.at[pl.ds(d0 * _DPS, rows * _DPS)],
            ttbuf.at[pl.ds(0, rows * _DPS)],
        )

        @pl.loop(0, 2 * _T, step=2)
        def _(k0):
            for h in range(2):
                k = k0 + h
                t = k // 2
                bh = b0 + (k % 2) * _BH
                ob = obufs[h]

                @pl.when(k0 >= 2)
                def _():
                    out_wait(h, rows)

                @pl.loop(0, _BH // 16, unroll=2)
                def _(g):
                    iv = idxbuf[t, pl.ds((k % 2) * _BH + g * 16, 16)]
                    for db in range(0, rows, 16):
                        w = min(16, rows - db)
                        vals = [
                            plsc.load_gather(ttbuf, [iv + (db + j) * _DPS])
                            for j in range(w)
                        ]
                        for j in range(w):
                            ob[db + j, pl.ds(g * 16, 16)] = vals[j]

                pltpu.make_async_copy(
                    ob.at[pl.ds(0, rows)],
                    out_hbm.at[t, pl.ds(d0, rows), pl.ds(bh, _BH)],
                    osems[h],
                ).start()

        out_wait(0, rows)
        out_wait(1, rows)

    @pl.loop(0, 3)
    def _(c):
        do_chunk(dbase + c * _RC, _RC)

    @pl.when(dblk < _NDB - 1)
    def _():
        do_chunk(dbase + 3 * _RC, _RC)

    @pl.when(dblk == _NDB - 1)
    def _():
        do_chunk(dbase + 3 * _RC, 8)


def kernel(idx, morphisms_logits):
    table = _sigmoid_table(morphisms_logits)
    tt_flat = jnp.pad(jnp.transpose(table), ((0, 0), (0, _DPS - _V))).reshape(-1)
    idxt = jnp.transpose(idx.astype(jnp.int32))
    out_t = _tgather_kernel(tt_flat, idxt)
    return jnp.transpose(out_t, (2, 0, 1))
